# baseline (speedup 1.0000x reference)
import functools

import jax
import jax.numpy as jnp
from jax import lax
from jax.experimental import pallas as pl
from jax.experimental.pallas import tpu as pltpu
from jax.experimental.pallas import tpu_sc as plsc

D_MODEL = 768
BATCH = 4
SEQ = 8192
NUM_TOKENS = BATCH * SEQ

_INFO = plsc.get_sparse_core_info()
NC = _INFO.num_cores
NS = _INFO.num_subcores
NW = NC * NS
T_PER_W = NUM_TOKENS // NW
K = 16
CHUNKS = T_PER_W // K
NBUF = 4
NVEC = D_MODEL // 16


@functools.partial(
    pl.kernel,
    out_type=jax.ShapeDtypeStruct((NUM_TOKENS, D_MODEL), jnp.float32),
    mesh=plsc.VectorSubcoreMesh(core_axis_name="c", subcore_axis_name="s"),
    scratch_types=[
        pltpu.VMEM((T_PER_W,), jnp.int32),
        pltpu.VMEM((NBUF, K, D_MODEL), jnp.float32),
        pltpu.VMEM((NBUF, K, D_MODEL), jnp.float32),
    ] + [pltpu.SemaphoreType.DMA] * (3 * NBUF),
)
def _lookup_add(x_hbm, idx_hbm, emb_hbm, out_hbm, idx_v, buf_e, buf_x, *sems):
    sem_x = sems[0:NBUF]
    sem_e = sems[NBUF:2 * NBUF]
    sem_o = sems[2 * NBUF:3 * NBUF]
    wid = lax.axis_index("s") * NC + lax.axis_index("c")
    base = wid * T_PER_W
    pltpu.sync_copy(idx_hbm.at[pl.ds(base, T_PER_W)], idx_v)

    def issue_in(c, b):
        row0 = base + c * K
        pltpu.async_copy(x_hbm.at[pl.ds(row0, K)], buf_x.at[b], sem_x[b])
        pltpu.async_copy(
            emb_hbm.at[idx_v.at[pl.ds(c * K, K)]], buf_e.at[b], sem_e[b])

    def wait_in(b):
        pltpu.make_async_copy(
            x_hbm.at[pl.ds(0, K)], buf_x.at[b], sem_x[b]).wait()
        pltpu.make_async_copy(
            emb_hbm.at[pl.ds(0, K)], buf_e.at[b], sem_e[b]).wait()

    def wait_out(b):
        pltpu.make_async_copy(
            buf_e.at[b], out_hbm.at[pl.ds(0, K)], sem_o[b]).wait()

    for c in range(NBUF - 1):
        issue_in(c, c)

    @pl.loop(0, CHUNKS, step=NBUF)
    def _group(cc):
        for t in range(NBUF):
            c = cc + t

            wait_in(t)

            @plsc.parallel_loop(0, K, 1, unroll=3)
            def _row(r):
                for j in range(NVEC):
                    plsc.addupdate(
                        buf_e.at[t, r, pl.ds(j * 16, 16)],
                        buf_x[t, r, pl.ds(j * 16, 16)])

            pltpu.async_copy(
                buf_e.at[t], out_hbm.at[pl.ds(base + c * K, K)], sem_o[t])

            nxt = (t + NBUF - 1) % NBUF

            @pl.when(c + NBUF - 1 < CHUNKS)
            def _prefetch():
                @pl.when(c >= 1)
                def _drain():
                    wait_out(nxt)

                issue_in(c + NBUF - 1, nxt)

    for t in range(NBUF):
        wait_out(t)


def kernel(x, frames_map, embeddings):
    x2 = x.reshape(NUM_TOKENS, D_MODEL)
    idx = frames_map.reshape(NUM_TOKENS).astype(jnp.int32)
    out = _lookup_add(x2, idx, embeddings)
    return out.reshape(BATCH, SEQ, D_MODEL)


# speedup vs baseline: 1.4223x; 1.4223x over previous
import functools

import jax
import jax.numpy as jnp
from jax import lax
from jax.experimental import pallas as pl
from jax.experimental.pallas import tpu as pltpu
from jax.experimental.pallas import tpu_sc as plsc

D_MODEL = 768
BATCH = 4
SEQ = 8192
NUM_TOKENS = BATCH * SEQ

_INFO = plsc.get_sparse_core_info()
NC = _INFO.num_cores
NS = _INFO.num_subcores
NW = NC * NS
T_PER_W = NUM_TOKENS // NW
K = 16
CHUNKS = T_PER_W // K
NBUF = 4
NVEC = D_MODEL // 16


@functools.partial(
    pl.kernel,
    out_type=jax.ShapeDtypeStruct((NUM_TOKENS, D_MODEL), jnp.float32),
    mesh=plsc.VectorSubcoreMesh(core_axis_name="c", subcore_axis_name="s"),
    scratch_types=[
        pltpu.VMEM((T_PER_W,), jnp.int32),
        pltpu.VMEM((NBUF, K, D_MODEL), jnp.float32),
        pltpu.VMEM((NBUF, K, D_MODEL), jnp.float32),
    ] + [pltpu.SemaphoreType.DMA] * (3 * NBUF),
)
def _lookup_add(x_hbm, idx_hbm, emb_hbm, out_hbm, idx_v, buf_e, buf_x, *sems):
    sem_x = sems[0:NBUF]
    sem_e = sems[NBUF:2 * NBUF]
    sem_o = sems[2 * NBUF:3 * NBUF]
    wid = lax.axis_index("s") * NC + lax.axis_index("c")
    base = wid * T_PER_W
    pltpu.sync_copy(idx_hbm.at[pl.ds(base, T_PER_W)], idx_v)

    def issue_in(c, b):
        row0 = base + c * K
        pltpu.async_copy(x_hbm.at[pl.ds(row0, K)], buf_x.at[b], sem_x[b])
        pltpu.async_copy(
            emb_hbm.at[idx_v.at[pl.ds(c * K, K)]], buf_e.at[b], sem_e[b])

    def wait_in(b):
        pltpu.make_async_copy(
            x_hbm.at[pl.ds(0, K)], buf_x.at[b], sem_x[b]).wait()
        pltpu.make_async_copy(
            emb_hbm.at[pl.ds(0, K)], buf_e.at[b], sem_e[b]).wait()

    def wait_out(b):
        pltpu.make_async_copy(
            buf_e.at[b], out_hbm.at[pl.ds(0, K)], sem_o[b]).wait()

    for c in range(NBUF - 1):
        issue_in(c, c)

    @pl.loop(0, CHUNKS, step=NBUF)
    def _group(cc):
        for t in range(NBUF):
            c = cc + t

            wait_in(t)

            @plsc.parallel_loop(0, K, 1, unroll=1)
            def _row(r):
                for j in range(NVEC):
                    plsc.addupdate(
                        buf_e.at[t, r, pl.ds(j * 16, 16)],
                        buf_x[t, r, pl.ds(j * 16, 16)])

            pltpu.async_copy(
                buf_e.at[t], out_hbm.at[pl.ds(base + c * K, K)], sem_o[t])

            nxt = (t + NBUF - 1) % NBUF

            @pl.when(c + NBUF - 1 < CHUNKS)
            def _prefetch():
                @pl.when(c >= 1)
                def _drain():
                    wait_out(nxt)

                issue_in(c + NBUF - 1, nxt)

    for t in range(NBUF):
        wait_out(t)


def kernel(x, frames_map, embeddings):
    x2 = x.reshape(NUM_TOKENS, D_MODEL)
    idx = frames_map.reshape(NUM_TOKENS).astype(jnp.int32)
    out = _lookup_add(x2, idx, embeddings)
    return out.reshape(BATCH, SEQ, D_MODEL)


# speedup vs baseline: 1.4383x; 1.0112x over previous
import functools

import jax
import jax.numpy as jnp
from jax import lax
from jax.experimental import pallas as pl
from jax.experimental.pallas import tpu as pltpu
from jax.experimental.pallas import tpu_sc as plsc

D_MODEL = 768
BATCH = 4
SEQ = 8192
NUM_TOKENS = BATCH * SEQ

_INFO = plsc.get_sparse_core_info()
NC = _INFO.num_cores
NS = _INFO.num_subcores
NW = NC * NS
T_PER_W = NUM_TOKENS // NW
K = 16
CHUNKS = T_PER_W // K
NBUF = 4
NVEC = D_MODEL // 16


@functools.partial(
    pl.kernel,
    out_type=jax.ShapeDtypeStruct((NUM_TOKENS, D_MODEL), jnp.float32),
    mesh=plsc.VectorSubcoreMesh(core_axis_name="c", subcore_axis_name="s"),
    scratch_types=[
        pltpu.VMEM((T_PER_W,), jnp.int32),
        pltpu.VMEM((NBUF, K, D_MODEL), jnp.float32),
        pltpu.VMEM((NBUF, K, D_MODEL), jnp.float32),
    ] + [pltpu.SemaphoreType.DMA] * (3 * NBUF),
)
def _lookup_add(x_hbm, idx_hbm, emb_hbm, out_hbm, idx_v, buf_e, buf_x, *sems):
    sem_x = sems[0:NBUF]
    sem_e = sems[NBUF:2 * NBUF]
    sem_o = sems[2 * NBUF:3 * NBUF]
    wid = lax.axis_index("s") * NC + lax.axis_index("c")
    base = wid * T_PER_W
    pltpu.sync_copy(idx_hbm.at[pl.ds(base, T_PER_W)], idx_v)

    def issue_in(c, b):
        row0 = base + c * K
        pltpu.async_copy(x_hbm.at[pl.ds(row0, K)], buf_x.at[b], sem_x[b])
        pltpu.async_copy(
            emb_hbm.at[idx_v.at[pl.ds(c * K, K)]], buf_e.at[b], sem_e[b])

    def wait_in(b):
        pltpu.make_async_copy(
            x_hbm.at[pl.ds(0, K)], buf_x.at[b], sem_x[b]).wait()
        pltpu.make_async_copy(
            emb_hbm.at[pl.ds(0, K)], buf_e.at[b], sem_e[b]).wait()

    def wait_out(b):
        pltpu.make_async_copy(
            buf_e.at[b], out_hbm.at[pl.ds(0, K)], sem_o[b]).wait()

    for c in range(NBUF - 1):
        issue_in(c, c)

    @pl.loop(0, CHUNKS, step=NBUF)
    def _group(cc):
        for t in range(NBUF):
            c = cc + t

            wait_in(t)

            @plsc.parallel_loop(0, K, 1, unroll=1)
            def _row(r):
                for j0 in range(0, NVEC, 8):
                    vals = [
                        buf_x[t, r, pl.ds((j0 + i) * 16, 16)] for i in range(8)
                    ]
                    for i in range(8):
                        plsc.addupdate(
                            buf_e.at[t, r, pl.ds((j0 + i) * 16, 16)], vals[i])

            pltpu.async_copy(
                buf_e.at[t], out_hbm.at[pl.ds(base + c * K, K)], sem_o[t])

            nxt = (t + NBUF - 1) % NBUF

            @pl.when(c + NBUF - 1 < CHUNKS)
            def _prefetch():
                @pl.when(c >= 1)
                def _drain():
                    wait_out(nxt)

                issue_in(c + NBUF - 1, nxt)

    for t in range(NBUF):
        wait_out(t)


def kernel(x, frames_map, embeddings):
    x2 = x.reshape(NUM_TOKENS, D_MODEL)
    idx = frames_map.reshape(NUM_TOKENS).astype(jnp.int32)
    out = _lookup_add(x2, idx, embeddings)
    return out.reshape(BATCH, SEQ, D_MODEL)


# speedup vs baseline: 1.4432x; 1.0034x over previous
import functools

import jax
import jax.numpy as jnp
from jax import lax
from jax.experimental import pallas as pl
from jax.experimental.pallas import tpu as pltpu
from jax.experimental.pallas import tpu_sc as plsc

D_MODEL = 768
BATCH = 4
SEQ = 8192
NUM_TOKENS = BATCH * SEQ

_INFO = plsc.get_sparse_core_info()
NC = _INFO.num_cores
NS = _INFO.num_subcores
NW = NC * NS
T_PER_W = NUM_TOKENS // NW
K = 16
CHUNKS = T_PER_W // K
NBUF = 4
NVEC = D_MODEL // 16


@functools.partial(
    pl.kernel,
    out_type=jax.ShapeDtypeStruct((NUM_TOKENS, D_MODEL), jnp.float32),
    mesh=plsc.VectorSubcoreMesh(core_axis_name="c", subcore_axis_name="s"),
    scratch_types=[
        pltpu.VMEM((T_PER_W,), jnp.int32),
        pltpu.VMEM((NBUF, K, D_MODEL), jnp.float32),
        pltpu.VMEM((NBUF, K, D_MODEL), jnp.float32),
    ] + [pltpu.SemaphoreType.DMA] * (3 * NBUF),
)
def _lookup_add(x_hbm, idx_hbm, emb_hbm, out_hbm, idx_v, buf_e, buf_x, *sems):
    sem_x = sems[0:NBUF]
    sem_e = sems[NBUF:2 * NBUF]
    sem_o = sems[2 * NBUF:3 * NBUF]
    wid = lax.axis_index("s") * NC + lax.axis_index("c")
    base = wid * T_PER_W
    pltpu.sync_copy(idx_hbm.at[pl.ds(base, T_PER_W)], idx_v)

    def issue_in(c, b):
        row0 = base + c * K
        pltpu.async_copy(x_hbm.at[pl.ds(row0, K)], buf_x.at[b], sem_x[b])
        pltpu.async_copy(
            emb_hbm.at[idx_v.at[pl.ds(c * K, K)]], buf_e.at[b], sem_e[b])

    def wait_in(b):
        pltpu.make_async_copy(
            x_hbm.at[pl.ds(0, K)], buf_x.at[b], sem_x[b]).wait()
        pltpu.make_async_copy(
            emb_hbm.at[pl.ds(0, K)], buf_e.at[b], sem_e[b]).wait()

    def wait_out(b):
        pltpu.make_async_copy(
            buf_e.at[b], out_hbm.at[pl.ds(0, K)], sem_o[b]).wait()

    for c in range(NBUF - 1):
        issue_in(c, c)

    @pl.loop(0, CHUNKS, step=NBUF)
    def _group(cc):
        for t in range(NBUF):
            c = cc + t

            wait_in(t)

            @plsc.parallel_loop(0, K, 1, unroll=1)
            def _row(r):
                for j0 in range(0, NVEC, 16):
                    vals = [
                        buf_x[t, r, pl.ds((j0 + i) * 16, 16)] for i in range(16)
                    ]
                    for i in range(16):
                        plsc.addupdate(
                            buf_e.at[t, r, pl.ds((j0 + i) * 16, 16)], vals[i])

            pltpu.async_copy(
                buf_e.at[t], out_hbm.at[pl.ds(base + c * K, K)], sem_o[t])

            nxt = (t + NBUF - 1) % NBUF

            @pl.when(c + NBUF - 1 < CHUNKS)
            def _prefetch():
                @pl.when(c >= 1)
                def _drain():
                    wait_out(nxt)

                issue_in(c + NBUF - 1, nxt)

    for t in range(NBUF):
        wait_out(t)


def kernel(x, frames_map, embeddings):
    x2 = x.reshape(NUM_TOKENS, D_MODEL)
    idx = frames_map.reshape(NUM_TOKENS).astype(jnp.int32)
    out = _lookup_add(x2, idx, embeddings)
    return out.reshape(BATCH, SEQ, D_MODEL)
